# baseline (device time: 28797 ns/iter reference)
import jax
import jax.numpy as jnp
from jax import lax
from jax.experimental import pallas as pl
from jax.experimental.pallas import tpu as pltpu

N_DEV = 4


def kernel(A, B):
    m, _ = A.shape
    _, n = B.shape

    def body(a_ref, b_ref, out_ref, comm_ref, send_sems, recv_sems):
        my_pos = lax.axis_index("i")
        left = (my_pos - 1) % N_DEV
        right = (my_pos + 1) % N_DEV

        barrier_sem = pltpu.get_barrier_semaphore()
        for nbr in [left, right]:
            pl.semaphore_signal(
                barrier_sem, inc=1,
                device_id=(nbr,), device_id_type=pl.DeviceIdType.MESH,
            )
        pl.semaphore_wait(barrier_sem, 2)

        partial = jnp.dot(
            a_ref[:, :].astype(jnp.bfloat16),
            b_ref[:, :].astype(jnp.bfloat16),
            preferred_element_type=jnp.float32,
        )
        out_ref[:, :] = partial
        comm_ref[0, :, :] = partial.astype(jnp.bfloat16)

        for h in range(N_DEV - 1):
            rdma = pltpu.make_async_remote_copy(
                src_ref=comm_ref.at[h],
                dst_ref=comm_ref.at[h + 1],
                send_sem=send_sems.at[h],
                recv_sem=recv_sems.at[h],
                device_id=(right,),
                device_id_type=pl.DeviceIdType.MESH,
            )
            rdma.start()
            rdma.wait()
            out_ref[:, :] += comm_ref[h + 1, :, :].astype(jnp.float32)

    return pl.pallas_call(
        body,
        out_shape=jax.ShapeDtypeStruct((m, n), jnp.float32),
        in_specs=[
            pl.BlockSpec(memory_space=pltpu.VMEM),
            pl.BlockSpec(memory_space=pltpu.VMEM),
        ],
        out_specs=pl.BlockSpec(memory_space=pltpu.VMEM),
        scratch_shapes=[
            pltpu.VMEM((N_DEV, m, n), jnp.bfloat16),
            pltpu.SemaphoreType.DMA((N_DEV - 1,)),
            pltpu.SemaphoreType.DMA((N_DEV - 1,)),
        ],
        compiler_params=pltpu.CompilerParams(collective_id=0),
    )(A, B)


# device time: 16283 ns/iter; 1.7685x vs baseline; 1.7685x over previous
import jax
import jax.numpy as jnp
from jax import lax
from jax.experimental import pallas as pl
from jax.experimental.pallas import tpu as pltpu

N_DEV = 4


def kernel(A, B):
    m, _ = A.shape
    _, n = B.shape
    qn = n // N_DEV

    def body(a_ref, b_ref, out_ref,
             part_full, rs_buf, ag_src, ag_buf,
             rs_send, rs_recv, ag_send, ag_recv):
        my = lax.axis_index("i")

        barrier_sem = pltpu.get_barrier_semaphore()
        for d in range(1, N_DEV):
            pl.semaphore_signal(
                barrier_sem, inc=1,
                device_id=((my + d) % N_DEV,),
                device_id_type=pl.DeviceIdType.MESH,
            )
        pl.semaphore_wait(barrier_sem, N_DEV - 1)

        partial = jnp.dot(
            a_ref[:, :].astype(jnp.bfloat16),
            b_ref[:, :].astype(jnp.bfloat16),
            preferred_element_type=jnp.float32,
        )

        part_full[:, :] = partial.astype(jnp.bfloat16)

        rs = []
        for d in range(1, N_DEV):
            q = (my + d) % N_DEV
            r = pltpu.make_async_remote_copy(
                src_ref=part_full.at[:, pl.ds(q * qn, qn)],
                dst_ref=rs_buf.at[d - 1],
                send_sem=rs_send.at[d - 1],
                recv_sem=rs_recv.at[d - 1],
                device_id=((my + d) % N_DEV,),
                device_id_type=pl.DeviceIdType.MESH,
            )
            r.start()
            rs.append(r)

        red = part_full[:, pl.ds(my * qn, qn)].astype(jnp.float32)

        for r in rs:
            r.wait_recv()
        red = red + (
            rs_buf[0].astype(jnp.float32)
            + rs_buf[1].astype(jnp.float32)
            + rs_buf[2].astype(jnp.float32)
        )

        out_ref[:, pl.ds(my * qn, qn)] = red
        ag_src[:, :] = red.astype(jnp.bfloat16)

        ag = []
        for d in range(1, N_DEV):
            r = pltpu.make_async_remote_copy(
                src_ref=ag_src,
                dst_ref=ag_buf.at[d - 1],
                send_sem=ag_send.at[d - 1],
                recv_sem=ag_recv.at[d - 1],
                device_id=((my + d) % N_DEV,),
                device_id_type=pl.DeviceIdType.MESH,
            )
            r.start()
            ag.append(r)

        for d, r in zip(range(1, N_DEV), ag):
            r.wait_recv()
            src_dev = (my - d) % N_DEV
            out_ref[:, pl.ds(src_dev * qn, qn)] = (
                ag_buf[d - 1].astype(jnp.float32)
            )

        for r in rs + ag:
            r.wait_send()

    return pl.pallas_call(
        body,
        out_shape=jax.ShapeDtypeStruct((m, n), jnp.float32),
        in_specs=[
            pl.BlockSpec(memory_space=pltpu.VMEM),
            pl.BlockSpec(memory_space=pltpu.VMEM),
        ],
        out_specs=pl.BlockSpec(memory_space=pltpu.VMEM),
        scratch_shapes=[
            pltpu.VMEM((m, n), jnp.bfloat16),
            pltpu.VMEM((N_DEV - 1, m, qn), jnp.bfloat16),
            pltpu.VMEM((m, qn), jnp.bfloat16),
            pltpu.VMEM((N_DEV - 1, m, qn), jnp.bfloat16),
            pltpu.SemaphoreType.DMA((N_DEV - 1,)),
            pltpu.SemaphoreType.DMA((N_DEV - 1,)),
            pltpu.SemaphoreType.DMA((N_DEV - 1,)),
            pltpu.SemaphoreType.DMA((N_DEV - 1,)),
        ],
        compiler_params=pltpu.CompilerParams(collective_id=0),
    )(A, B)


# device time: 14959 ns/iter; 1.9251x vs baseline; 1.0885x over previous
import jax
import jax.numpy as jnp
from jax import lax
from jax.experimental import pallas as pl
from jax.experimental.pallas import tpu as pltpu

N_DEV = 4
N_HALF = 2


def kernel(A, B):
    m, _ = A.shape
    _, n = B.shape
    qn = n // N_DEV
    mh = m // N_HALF

    def body(a_ref, b_ref, out_ref,
             part_full, rs_buf, ag_src, ag_buf,
             rs_send, rs_recv, ag_send, ag_recv):
        my = lax.axis_index("i")

        barrier_sem = pltpu.get_barrier_semaphore()
        for d in range(1, N_DEV):
            pl.semaphore_signal(
                barrier_sem, inc=1,
                device_id=((my + d) % N_DEV,),
                device_id_type=pl.DeviceIdType.MESH,
            )
        pl.semaphore_wait(barrier_sem, N_DEV - 1)

        def start_rs(h):
            sends = []
            for d in range(1, N_DEV):
                q = (my + d) % N_DEV
                r = pltpu.make_async_remote_copy(
                    src_ref=part_full.at[pl.ds(h * mh, mh), pl.ds(q * qn, qn)],
                    dst_ref=rs_buf.at[d - 1, h],
                    send_sem=rs_send.at[d - 1, h],
                    recv_sem=rs_recv.at[d - 1, h],
                    device_id=((my + d) % N_DEV,),
                    device_id_type=pl.DeviceIdType.MESH,
                )
                r.start()
                sends.append(r)
            return sends

        def start_ag(h):
            sends = []
            for d in range(1, N_DEV):
                r = pltpu.make_async_remote_copy(
                    src_ref=ag_src.at[h],
                    dst_ref=ag_buf.at[d - 1, h],
                    send_sem=ag_send.at[d - 1, h],
                    recv_sem=ag_recv.at[d - 1, h],
                    device_id=((my + d) % N_DEV,),
                    device_id_type=pl.DeviceIdType.MESH,
                )
                r.start()
                sends.append(r)
            return sends

        def reduce_and_ag(h, rs_sends):
            for r in rs_sends:
                r.wait_recv()
            red = part_full[pl.ds(h * mh, mh), pl.ds(my * qn, qn)].astype(
                jnp.float32
            )
            red = red + (
                rs_buf[0, h].astype(jnp.float32)
                + rs_buf[1, h].astype(jnp.float32)
                + rs_buf[2, h].astype(jnp.float32)
            )
            out_ref[pl.ds(h * mh, mh), pl.ds(my * qn, qn)] = red
            ag_src[h] = red.astype(jnp.bfloat16)
            return start_ag(h)

        ab = a_ref[:, :].astype(jnp.bfloat16)
        bb = b_ref[:, :].astype(jnp.bfloat16)

        part_full[0:mh, :] = jnp.dot(
            ab[0:mh, :], bb, preferred_element_type=jnp.float32
        ).astype(jnp.bfloat16)
        rs0 = start_rs(0)

        part_full[mh:m, :] = jnp.dot(
            ab[mh:m, :], bb, preferred_element_type=jnp.float32
        ).astype(jnp.bfloat16)
        rs1 = start_rs(1)

        ag0 = reduce_and_ag(0, rs0)
        ag1 = reduce_and_ag(1, rs1)

        for h, ags in ((0, ag0), (1, ag1)):
            for d, r in zip(range(1, N_DEV), ags):
                r.wait_recv()
                src_dev = (my - d) % N_DEV
                out_ref[pl.ds(h * mh, mh), pl.ds(src_dev * qn, qn)] = (
                    ag_buf[d - 1, h].astype(jnp.float32)
                )

        for r in rs0 + rs1 + ag0 + ag1:
            r.wait_send()

    return pl.pallas_call(
        body,
        out_shape=jax.ShapeDtypeStruct((m, n), jnp.float32),
        in_specs=[
            pl.BlockSpec(memory_space=pltpu.VMEM),
            pl.BlockSpec(memory_space=pltpu.VMEM),
        ],
        out_specs=pl.BlockSpec(memory_space=pltpu.VMEM),
        scratch_shapes=[
            pltpu.VMEM((m, n), jnp.bfloat16),
            pltpu.VMEM((N_DEV - 1, N_HALF, mh, qn), jnp.bfloat16),
            pltpu.VMEM((N_HALF, mh, qn), jnp.bfloat16),
            pltpu.VMEM((N_DEV - 1, N_HALF, mh, qn), jnp.bfloat16),
            pltpu.SemaphoreType.DMA((N_DEV - 1, N_HALF)),
            pltpu.SemaphoreType.DMA((N_DEV - 1, N_HALF)),
            pltpu.SemaphoreType.DMA((N_DEV - 1, N_HALF)),
            pltpu.SemaphoreType.DMA((N_DEV - 1, N_HALF)),
        ],
        compiler_params=pltpu.CompilerParams(collective_id=0),
    )(A, B)


# device time: 3762 ns/iter; 7.6547x vs baseline; 3.9763x over previous
import jax
import jax.numpy as jnp
from jax import lax
from jax.experimental import pallas as pl
from jax.experimental.pallas import tpu as pltpu

N_DEV = 4
N_CHUNK = 4


def kernel(A, B):
    m, _ = A.shape
    _, n = B.shape
    qn = n // N_DEV
    mc = m // N_CHUNK

    def body(a_ref, b_ref, out_ref,
             part_full, rs_buf, ag_src, ag_buf,
             rs_send, rs_recv, ag_send, ag_recv):
        my = lax.axis_index("i")

        barrier_sem = pltpu.get_barrier_semaphore()
        for d in range(1, N_DEV):
            pl.semaphore_signal(
                barrier_sem, inc=1,
                device_id=((my + d) % N_DEV,),
                device_id_type=pl.DeviceIdType.MESH,
            )
        pl.semaphore_wait(barrier_sem, N_DEV - 1)

        def start_rs(c):
            sends = []
            for d in range(1, N_DEV):
                q = (my + d) % N_DEV
                r = pltpu.make_async_remote_copy(
                    src_ref=part_full.at[pl.ds(c * mc, mc), pl.ds(q * qn, qn)],
                    dst_ref=rs_buf.at[d - 1, c],
                    send_sem=rs_send.at[d - 1, c],
                    recv_sem=rs_recv.at[d - 1, c],
                    device_id=((my + d) % N_DEV,),
                    device_id_type=pl.DeviceIdType.MESH,
                )
                r.start()
                sends.append(r)
            return sends

        def start_ag(c):
            sends = []
            for d in range(1, N_DEV):
                r = pltpu.make_async_remote_copy(
                    src_ref=ag_src.at[c],
                    dst_ref=ag_buf.at[d - 1, c],
                    send_sem=ag_send.at[d - 1, c],
                    recv_sem=ag_recv.at[d - 1, c],
                    device_id=((my + d) % N_DEV,),
                    device_id_type=pl.DeviceIdType.MESH,
                )
                r.start()
                sends.append(r)
            return sends

        def reduce_and_ag(c, rs_sends):
            for r in rs_sends:
                r.wait_recv()
            red = part_full[pl.ds(c * mc, mc), pl.ds(my * qn, qn)].astype(
                jnp.float32
            )
            red = red + (
                rs_buf[0, c].astype(jnp.float32)
                + rs_buf[1, c].astype(jnp.float32)
                + rs_buf[2, c].astype(jnp.float32)
            )
            out_ref[pl.ds(c * mc, mc), pl.ds(my * qn, qn)] = red
            ag_src[c] = red.astype(jnp.bfloat16)
            return start_ag(c)

        ab = a_ref[:, :].astype(jnp.bfloat16)
        bb = b_ref[:, :].astype(jnp.bfloat16)

        rs = []
        for c in range(N_CHUNK):
            part_full[c * mc:(c + 1) * mc, :] = jnp.dot(
                ab[c * mc:(c + 1) * mc, :], bb,
                preferred_element_type=jnp.float32,
            ).astype(jnp.bfloat16)
            rs.append(start_rs(c))

        ag = [reduce_and_ag(c, rs[c]) for c in range(N_CHUNK)]

        for c in range(N_CHUNK):
            for d, r in zip(range(1, N_DEV), ag[c]):
                r.wait_recv()
                src_dev = (my - d) % N_DEV
                out_ref[pl.ds(c * mc, mc), pl.ds(src_dev * qn, qn)] = (
                    ag_buf[d - 1, c].astype(jnp.float32)
                )

        for sends in rs + ag:
            for r in sends:
                r.wait_send()

    return pl.pallas_call(
        body,
        out_shape=jax.ShapeDtypeStruct((m, n), jnp.float32),
        in_specs=[
            pl.BlockSpec(memory_space=pltpu.VMEM),
            pl.BlockSpec(memory_space=pltpu.VMEM),
        ],
        out_specs=pl.BlockSpec(memory_space=pltpu.VMEM),
        scratch_shapes=[
            pltpu.VMEM((m, n), jnp.bfloat16),
            pltpu.VMEM((N_DEV - 1, N_CHUNK, mc, qn), jnp.bfloat16),
            pltpu.VMEM((N_CHUNK, mc, qn), jnp.bfloat16),
            pltpu.VMEM((N_DEV - 1, N_CHUNK, mc, qn), jnp.bfloat16),
            pltpu.SemaphoreType.DMA((N_DEV - 1, N_CHUNK)),
            pltpu.SemaphoreType.DMA((N_DEV - 1, N_CHUNK)),
            pltpu.SemaphoreType.DMA((N_DEV - 1, N_CHUNK)),
            pltpu.SemaphoreType.DMA((N_DEV - 1, N_CHUNK)),
        ],
        compiler_params=pltpu.CompilerParams(collective_id=0),
    )(A, B)
